# baseline (speedup 1.0000x reference)
import functools

import jax
import jax.numpy as jnp
from jax import lax
from jax.experimental import pallas as pl
from jax.experimental.pallas import tpu as pltpu
from jax.experimental.pallas import tpu_sc as plsc

N = 10000
E = 320000
F = 128
A = 16
OUT = 256
LANES = 128
E_PAD = 327680
SLABS = E_PAD // LANES
NW = 32
WSLABS = SLABS // NW
G = 4
NG = WSLABS // G
N_ACC = N + 16
TILE = 2560
EPS = 1e-5

_MESH = plsc.VectorSubcoreMesh(core_axis_name="c", subcore_axis_name="s")


GB = 2
NGB = WSLABS // GB


def _gather_body(x_hbm, gsrc_hbm, gdst_hbm, xi_hbm, xj_hbm, idx_v, rows_v,
                 gsem0, gsem1, wsem0, wsem1):
    wid = lax.axis_index("s") * 2 + lax.axis_index("c")
    gsems = (gsem0, gsem1)
    wsems = (wsem0, wsem1)

    def do_one(idx2d, out_hbm):
        base = wid * WSLABS

        def load_fire(g, b):
            pltpu.sync_copy(idx2d.at[pl.ds(base + g * GB, GB)], idx_v.at[b])
            for j in range(GB):
                pltpu.async_copy(
                    x_hbm.at[idx_v.at[b, j]],
                    rows_v.at[b, pl.ds(j * LANES, LANES)],
                    gsems[b],
                )

        def wait_gather(b):
            for j in range(GB):
                pltpu.make_async_copy(
                    x_hbm.at[idx_v.at[b, j]],
                    rows_v.at[b, pl.ds(j * LANES, LANES)],
                    gsems[b],
                ).wait()

        def out_slice(g):
            return out_hbm.at[pl.ds((base + g * GB) * LANES, GB * LANES)]

        load_fire(0, 0)

        def body(k, carry):
            for b in range(2):
                g = k * 2 + b
                wait_gather(b)

                @pl.when(g >= 1)
                def _():
                    pltpu.make_async_copy(
                        rows_v.at[1 - b], out_slice(g - 1), wsems[1 - b]
                    ).wait()

                @pl.when(g + 1 < NGB)
                def _():
                    load_fire(g + 1, 1 - b)

                pltpu.async_copy(rows_v.at[b], out_slice(g), wsems[b])
            return carry

        lax.fori_loop(0, NGB // 2, body, 0)
        pltpu.make_async_copy(
            rows_v.at[1], out_slice(NGB - 1), wsems[1]).wait()

    do_one(gsrc_hbm, xi_hbm)
    do_one(gdst_hbm, xj_hbm)


_gather_call = functools.partial(
    pl.kernel,
    out_type=(
        jax.ShapeDtypeStruct((E_PAD, F), jnp.float32),
        jax.ShapeDtypeStruct((E_PAD, F), jnp.float32),
    ),
    mesh=_MESH,
    scratch_types=[
        pltpu.VMEM((2, GB, LANES), jnp.int32),
        pltpu.VMEM((2, GB * LANES, F), jnp.float32),
        pltpu.SemaphoreType.DMA,
        pltpu.SemaphoreType.DMA,
        pltpu.SemaphoreType.DMA,
        pltpu.SemaphoreType.DMA,
    ],
)(_gather_body)


def _stats_body(xi_ref, xj_ref, ea_ref, w1_ref, w2_ref, w3_ref, b_ref,
                s_ref, ss_ref):
    z = (
        jnp.dot(xi_ref[...], w1_ref[...], preferred_element_type=jnp.float32)
        + jnp.dot(xj_ref[...], w2_ref[...], preferred_element_type=jnp.float32)
        + jnp.dot(ea_ref[...], w3_ref[...], preferred_element_type=jnp.float32)
        + b_ref[0:1, :]
    )
    ones8 = jnp.ones((8, TILE), jnp.float32)
    s8 = jnp.dot(ones8, z, preferred_element_type=jnp.float32)
    ss8 = jnp.dot(ones8, z * z, preferred_element_type=jnp.float32)

    @pl.when(pl.program_id(0) == 0)
    def _():
        s_ref[...] = jnp.zeros_like(s_ref)
        ss_ref[...] = jnp.zeros_like(ss_ref)

    s_ref[...] += s8
    ss_ref[...] += ss8


def _stats_call(xi, xj, ea, w1, w2, w3, bb):
    return pl.pallas_call(
        _stats_body,
        grid=(E // TILE,),
        in_specs=[
            pl.BlockSpec((TILE, F), lambda i: (i, 0)),
            pl.BlockSpec((TILE, F), lambda i: (i, 0)),
            pl.BlockSpec((TILE, A), lambda i: (i, 0)),
            pl.BlockSpec((F, OUT), lambda i: (0, 0)),
            pl.BlockSpec((F, OUT), lambda i: (0, 0)),
            pl.BlockSpec((A, OUT), lambda i: (0, 0)),
            pl.BlockSpec((8, OUT), lambda i: (0, 0)),
        ],
        out_specs=[
            pl.BlockSpec((8, OUT), lambda i: (0, 0)),
            pl.BlockSpec((8, OUT), lambda i: (0, 0)),
        ],
        out_shape=[
            jax.ShapeDtypeStruct((8, OUT), jnp.float32),
            jax.ShapeDtypeStruct((8, OUT), jnp.float32),
        ],
        compiler_params=pltpu.CompilerParams(
            dimension_semantics=("arbitrary",)),
    )(xi, xj, ea, w1, w2, w3, bb)


def _msg_body(xi_ref, xj_ref, ea_ref, w1_ref, w2_ref, w3_ref, b_ref, msg_ref):
    z = (
        jnp.dot(xi_ref[...], w1_ref[...], preferred_element_type=jnp.float32)
        + jnp.dot(xj_ref[...], w2_ref[...], preferred_element_type=jnp.float32)
        + jnp.dot(ea_ref[...], w3_ref[...], preferred_element_type=jnp.float32)
        + b_ref[0:1, :]
    )
    filt = jax.nn.sigmoid(z[:, :F])
    core = jax.nn.softplus(z[:, F:])
    msg_ref[...] = filt * core


def _msg_call(xi, xj, ea, w1f, w2f, w3f, bbf):
    return pl.pallas_call(
        _msg_body,
        grid=(E_PAD // TILE,),
        in_specs=[
            pl.BlockSpec((TILE, F), lambda i: (i, 0)),
            pl.BlockSpec((TILE, F), lambda i: (i, 0)),
            pl.BlockSpec((TILE, A), lambda i: (i, 0)),
            pl.BlockSpec((F, OUT), lambda i: (0, 0)),
            pl.BlockSpec((F, OUT), lambda i: (0, 0)),
            pl.BlockSpec((A, OUT), lambda i: (0, 0)),
            pl.BlockSpec((8, OUT), lambda i: (0, 0)),
        ],
        out_specs=pl.BlockSpec((TILE, F), lambda i: (i, 0)),
        out_shape=jax.ShapeDtypeStruct((E_PAD, F), jnp.float32),
        compiler_params=pltpu.CompilerParams(
            dimension_semantics=("parallel",)),
    )(xi, xj, ea, w1f, w2f, w3f, bbf)


NG2 = WSLABS


def _scatter_body(msg_hbm, ssrc_hbm, zero_hbm, out_hbm, idx_v, rows_v, acc,
                  lsem0, lsem1):
    cid = lax.axis_index("c")
    sid = lax.axis_index("s")
    wid = sid * 2 + cid
    lsems = (lsem0, lsem1)
    base = wid * WSLABS

    @pl.when(sid == 0)
    def _():
        pltpu.sync_copy(zero_hbm, acc)

    plsc.subcore_barrier()

    def load_fire(g, b):
        pltpu.sync_copy(ssrc_hbm.at[pl.ds(base + g, 1)], idx_v.at[b])
        pltpu.async_copy(
            msg_hbm.at[pl.ds((base + g) * LANES, LANES)],
            rows_v.at[b],
            lsems[b],
        )

    load_fire(0, 0)

    def body(k, carry):
        for b in range(2):
            g = k * 2 + b
            pltpu.make_async_copy(
                msg_hbm.at[pl.ds((base + g) * LANES, LANES)],
                rows_v.at[b],
                lsems[b],
            ).wait()

            @pl.when(g + 1 < NG2)
            def _():
                load_fire(g + 1, 1 - b)

            pltpu.sync_copy(rows_v.at[b], acc.at[idx_v.at[b, 0]], add=True)
        return carry

    lax.fori_loop(0, NG2 // 2, body, 0)
    plsc.subcore_barrier()

    @pl.when(sid == 0)
    def _():
        pltpu.sync_copy(acc.at[pl.ds(0, N)], out_hbm.at[cid])


_scatter_call = functools.partial(
    pl.kernel,
    out_type=jax.ShapeDtypeStruct((2, N, F), jnp.float32),
    mesh=_MESH,
    scratch_types=[
        pltpu.VMEM((2, 1, LANES), jnp.int32),
        pltpu.VMEM((2, LANES, F), jnp.float32),
        pltpu.VMEM_SHARED((N_ACC, F), jnp.float32),
        pltpu.SemaphoreType.DMA,
        pltpu.SemaphoreType.DMA,
    ],
)(_scatter_body)


def _bn2_body(p_ref, g2_ref, b2_ref, out_ref):
    zsum = p_ref[0] + p_ref[1]
    mu = jnp.mean(zsum, axis=0, keepdims=True)
    var = jnp.mean((zsum - mu) ** 2, axis=0, keepdims=True)
    out_ref[...] = (zsum - mu) * lax.rsqrt(var + EPS) * g2_ref[0:1, :] + b2_ref[0:1, :]


def _bn2_call(partials, g2, b2):
    return pl.pallas_call(
        _bn2_body,
        out_shape=jax.ShapeDtypeStruct((N, F), jnp.float32),
    )(partials, g2, b2)


def kernel(x, edge_index, edge_attr, W, b, gamma1, beta1, gamma2, beta2):
    src = edge_index[0].astype(jnp.int32)
    dst = edge_index[1].astype(jnp.int32)
    npad = E_PAD - E

    pad_g = jnp.arange(npad, dtype=jnp.int32) % N
    gsrc = jnp.concatenate([src, pad_g]).reshape(SLABS, LANES)
    gdst = jnp.concatenate([dst, pad_g]).reshape(SLABS, LANES)
    pad_s = N + (jnp.arange(npad, dtype=jnp.int32) % 16)
    ssrc = jnp.concatenate([src, pad_s]).reshape(SLABS, LANES)
    ea_pad = jnp.concatenate(
        [edge_attr, jnp.zeros((npad, A), jnp.float32)], axis=0)

    w1 = W[:F]
    w2 = W[F:2 * F]
    w3 = W[2 * F:]
    bb = jnp.broadcast_to(b, (8, OUT))

    xi, xj = _gather_call(x, gsrc, gdst)

    s8, ss8 = _stats_call(xi, xj, ea_pad, w1, w2, w3, bb)
    s = s8[0]
    ss = ss8[0]
    mu = s / E
    var = ss / E - mu * mu
    scale1 = gamma1 * lax.rsqrt(var + EPS)
    shift1 = beta1 - mu * scale1
    w1f = w1 * scale1
    w2f = w2 * scale1
    w3f = w3 * scale1
    bbf = jnp.broadcast_to(b * scale1 + shift1, (8, OUT))

    msg = _msg_call(xi, xj, ea_pad, w1f, w2f, w3f, bbf)

    zero = jnp.zeros((N_ACC, F), jnp.float32)
    summed = _scatter_call(msg, ssrc, zero)

    g2 = jnp.broadcast_to(gamma2, (8, F))
    b2 = jnp.broadcast_to(beta2, (8, F))
    return _bn2_call(summed, g2, b2)


# speedup vs baseline: 1.1084x; 1.1084x over previous
import functools

import jax
import jax.numpy as jnp
from jax import lax
from jax.experimental import pallas as pl
from jax.experimental.pallas import tpu as pltpu
from jax.experimental.pallas import tpu_sc as plsc

N = 10000
E = 320000
F = 128
A = 16
OUT = 256
LANES = 128
EPS = 1e-5

NCHUNK = 2
EC = E // NCHUNK
EC_PAD = 163840
SLABS_C = EC_PAD // LANES
NW = 32
WSLABS = SLABS_C // NW
GB = 2
NGB = WSLABS // GB
NSC = WSLABS
N_ACC = N + 16
T2 = 3200
T3 = 2560

_MESH = plsc.VectorSubcoreMesh(core_axis_name="c", subcore_axis_name="s")


def _gather_body(x_hbm, gsrc_hbm, gdst_hbm, xi_hbm, xj_hbm, idx_v, rows_v,
                 gsem0, gsem1, wsem0, wsem1):
    wid = lax.axis_index("s") * 2 + lax.axis_index("c")
    gsems = (gsem0, gsem1)
    wsems = (wsem0, wsem1)

    def do_one(idx2d, out_hbm):
        base = wid * WSLABS

        def load_fire(g, b):
            pltpu.sync_copy(idx2d.at[pl.ds(base + g * GB, GB)], idx_v.at[b])
            for j in range(GB):
                pltpu.async_copy(
                    x_hbm.at[idx_v.at[b, j]],
                    rows_v.at[b, pl.ds(j * LANES, LANES)],
                    gsems[b],
                )

        def wait_gather(b):
            for j in range(GB):
                pltpu.make_async_copy(
                    x_hbm.at[idx_v.at[b, j]],
                    rows_v.at[b, pl.ds(j * LANES, LANES)],
                    gsems[b],
                ).wait()

        def out_slice(g):
            return out_hbm.at[pl.ds((base + g * GB) * LANES, GB * LANES)]

        load_fire(0, 0)

        def body(k, carry):
            for b in range(2):
                g = k * 2 + b
                wait_gather(b)

                @pl.when(g >= 1)
                def _():
                    pltpu.make_async_copy(
                        rows_v.at[1 - b], out_slice(g - 1), wsems[1 - b]
                    ).wait()

                @pl.when(g + 1 < NGB)
                def _():
                    load_fire(g + 1, 1 - b)

                pltpu.async_copy(rows_v.at[b], out_slice(g), wsems[b])
            return carry

        lax.fori_loop(0, NGB // 2, body, 0)
        pltpu.make_async_copy(
            rows_v.at[1], out_slice(NGB - 1), wsems[1]).wait()

    do_one(gsrc_hbm, xi_hbm)
    do_one(gdst_hbm, xj_hbm)


_gather_call = functools.partial(
    pl.kernel,
    out_type=(
        jax.ShapeDtypeStruct((EC_PAD, F), jnp.float32),
        jax.ShapeDtypeStruct((EC_PAD, F), jnp.float32),
    ),
    mesh=_MESH,
    scratch_types=[
        pltpu.VMEM((2, GB, LANES), jnp.int32),
        pltpu.VMEM((2, GB * LANES, F), jnp.float32),
        pltpu.SemaphoreType.DMA,
        pltpu.SemaphoreType.DMA,
        pltpu.SemaphoreType.DMA,
        pltpu.SemaphoreType.DMA,
    ],
)(_gather_body)


def _stats_body(xi_ref, xj_ref, ea_ref, w1_ref, w2_ref, w3_ref, b_ref,
                s_ref, ss_ref):
    z = (
        jnp.dot(xi_ref[...], w1_ref[...], preferred_element_type=jnp.float32)
        + jnp.dot(xj_ref[...], w2_ref[...], preferred_element_type=jnp.float32)
        + jnp.dot(ea_ref[...], w3_ref[...], preferred_element_type=jnp.float32)
        + b_ref[0:1, :]
    )
    ones8 = jnp.ones((8, T2), jnp.float32)
    s8 = jnp.dot(ones8, z, preferred_element_type=jnp.float32)
    ss8 = jnp.dot(ones8, z * z, preferred_element_type=jnp.float32)

    @pl.when(pl.program_id(0) == 0)
    def _():
        s_ref[...] = jnp.zeros_like(s_ref)
        ss_ref[...] = jnp.zeros_like(ss_ref)

    s_ref[...] += s8
    ss_ref[...] += ss8


def _stats_call(xi, xj, ea, w1, w2, w3, bb):
    return pl.pallas_call(
        _stats_body,
        grid=(EC // T2,),
        in_specs=[
            pl.BlockSpec((T2, F), lambda i: (i, 0)),
            pl.BlockSpec((T2, F), lambda i: (i, 0)),
            pl.BlockSpec((T2, A), lambda i: (i, 0)),
            pl.BlockSpec((F, OUT), lambda i: (0, 0)),
            pl.BlockSpec((F, OUT), lambda i: (0, 0)),
            pl.BlockSpec((A, OUT), lambda i: (0, 0)),
            pl.BlockSpec((8, OUT), lambda i: (0, 0)),
        ],
        out_specs=[
            pl.BlockSpec((8, OUT), lambda i: (0, 0)),
            pl.BlockSpec((8, OUT), lambda i: (0, 0)),
        ],
        out_shape=[
            jax.ShapeDtypeStruct((8, OUT), jnp.float32),
            jax.ShapeDtypeStruct((8, OUT), jnp.float32),
        ],
        compiler_params=pltpu.CompilerParams(
            dimension_semantics=("arbitrary",)),
    )(xi, xj, ea, w1, w2, w3, bb)


def _msg_body(xi_ref, xj_ref, ea_ref, w1_ref, w2_ref, w3_ref, b_ref, msg_ref):
    z = (
        jnp.dot(xi_ref[...], w1_ref[...], preferred_element_type=jnp.float32)
        + jnp.dot(xj_ref[...], w2_ref[...], preferred_element_type=jnp.float32)
        + jnp.dot(ea_ref[...], w3_ref[...], preferred_element_type=jnp.float32)
        + b_ref[0:1, :]
    )
    filt = jax.nn.sigmoid(z[:, :F])
    core = jax.nn.softplus(z[:, F:])
    msg_ref[...] = filt * core


def _msg_call(xi, xj, ea, w1f, w2f, w3f, bbf):
    return pl.pallas_call(
        _msg_body,
        grid=(EC_PAD // T3,),
        in_specs=[
            pl.BlockSpec((T3, F), lambda i: (i, 0)),
            pl.BlockSpec((T3, F), lambda i: (i, 0)),
            pl.BlockSpec((T3, A), lambda i: (i, 0)),
            pl.BlockSpec((F, OUT), lambda i: (0, 0)),
            pl.BlockSpec((F, OUT), lambda i: (0, 0)),
            pl.BlockSpec((A, OUT), lambda i: (0, 0)),
            pl.BlockSpec((8, OUT), lambda i: (0, 0)),
        ],
        out_specs=pl.BlockSpec((T3, F), lambda i: (i, 0)),
        out_shape=jax.ShapeDtypeStruct((EC_PAD, F), jnp.float32),
        compiler_params=pltpu.CompilerParams(
            dimension_semantics=("parallel",)),
    )(xi, xj, ea, w1f, w2f, w3f, bbf)


def _scatter_body(msg_hbm, ssrc_hbm, zero_hbm, out_hbm, idx_v, rows_v, acc,
                  lsem0, lsem1):
    cid = lax.axis_index("c")
    sid = lax.axis_index("s")
    wid = sid * 2 + cid
    lsems = (lsem0, lsem1)
    base = wid * WSLABS

    @pl.when(sid == 0)
    def _():
        pltpu.sync_copy(zero_hbm, acc)

    plsc.subcore_barrier()

    def load_fire(g, b):
        pltpu.sync_copy(ssrc_hbm.at[pl.ds(base + g, 1)], idx_v.at[b])
        pltpu.async_copy(
            msg_hbm.at[pl.ds((base + g) * LANES, LANES)],
            rows_v.at[b],
            lsems[b],
        )

    load_fire(0, 0)

    def body(k, carry):
        for b in range(2):
            g = k * 2 + b
            pltpu.make_async_copy(
                msg_hbm.at[pl.ds((base + g) * LANES, LANES)],
                rows_v.at[b],
                lsems[b],
            ).wait()

            @pl.when(g + 1 < NSC)
            def _():
                load_fire(g + 1, 1 - b)

            pltpu.sync_copy(rows_v.at[b], acc.at[idx_v.at[b, 0]], add=True)
        return carry

    lax.fori_loop(0, NSC // 2, body, 0)
    plsc.subcore_barrier()

    @pl.when(sid == 0)
    def _():
        pltpu.sync_copy(acc.at[pl.ds(0, N)], out_hbm.at[cid])


_scatter_call = functools.partial(
    pl.kernel,
    out_type=jax.ShapeDtypeStruct((2, N, F), jnp.float32),
    mesh=_MESH,
    scratch_types=[
        pltpu.VMEM((2, 1, LANES), jnp.int32),
        pltpu.VMEM((2, LANES, F), jnp.float32),
        pltpu.VMEM_SHARED((N_ACC, F), jnp.float32),
        pltpu.SemaphoreType.DMA,
        pltpu.SemaphoreType.DMA,
    ],
)(_scatter_body)


def _bn2_body(pa_ref, pb_ref, g2_ref, b2_ref, out_ref):
    zsum = (pa_ref[0] + pa_ref[1]) + (pb_ref[0] + pb_ref[1])
    mu = jnp.mean(zsum, axis=0, keepdims=True)
    var = jnp.mean((zsum - mu) ** 2, axis=0, keepdims=True)
    out_ref[...] = (zsum - mu) * lax.rsqrt(var + EPS) * g2_ref[0:1, :] + b2_ref[0:1, :]


def _bn2_call(pa, pb, g2, b2):
    return pl.pallas_call(
        _bn2_body,
        out_shape=jax.ShapeDtypeStruct((N, F), jnp.float32),
    )(pa, pb, g2, b2)


def _chunk_indices(src_c, dst_c):
    npad = EC_PAD - EC
    pad_g = jnp.arange(npad, dtype=jnp.int32) % N
    gsrc = jnp.concatenate([src_c, pad_g]).reshape(SLABS_C, LANES)
    gdst = jnp.concatenate([dst_c, pad_g]).reshape(SLABS_C, LANES)
    pad_s = N + (jnp.arange(npad, dtype=jnp.int32) % 16)
    ssrc = jnp.concatenate([src_c, pad_s]).reshape(SLABS_C, LANES)
    return gsrc, gdst, ssrc


def kernel(x, edge_index, edge_attr, W, b, gamma1, beta1, gamma2, beta2):
    src = edge_index[0].astype(jnp.int32)
    dst = edge_index[1].astype(jnp.int32)
    npad = EC_PAD - EC

    w1 = W[:F]
    w2 = W[F:2 * F]
    w3 = W[2 * F:]
    bb = jnp.broadcast_to(b, (8, OUT))
    zero_pad_ea = jnp.zeros((npad, A), jnp.float32)

    chunks = []
    for ci in range(NCHUNK):
        lo = ci * EC
        gsrc, gdst, ssrc = _chunk_indices(
            lax.dynamic_slice_in_dim(src, lo, EC),
            lax.dynamic_slice_in_dim(dst, lo, EC))
        ea_c = jnp.concatenate(
            [lax.dynamic_slice_in_dim(edge_attr, lo, EC), zero_pad_ea], axis=0)
        xi, xj = _gather_call(x, gsrc, gdst)
        s8, ss8 = _stats_call(xi, xj, ea_c, w1, w2, w3, bb)
        chunks.append((xi, xj, ea_c, ssrc, s8[0], ss8[0]))

    s = chunks[0][4] + chunks[1][4]
    ss = chunks[0][5] + chunks[1][5]
    mu = s / E
    var = ss / E - mu * mu
    scale1 = gamma1 * lax.rsqrt(var + EPS)
    shift1 = beta1 - mu * scale1
    w1f = w1 * scale1
    w2f = w2 * scale1
    w3f = w3 * scale1
    bbf = jnp.broadcast_to(b * scale1 + shift1, (8, OUT))

    zero = jnp.zeros((N_ACC, F), jnp.float32)
    parts = []
    for (xi, xj, ea_c, ssrc, _, _) in chunks:
        msg = _msg_call(xi, xj, ea_c, w1f, w2f, w3f, bbf)
        parts.append(_scatter_call(msg, ssrc, zero))

    g2 = jnp.broadcast_to(gamma2, (8, F))
    b2 = jnp.broadcast_to(beta2, (8, F))
    return _bn2_call(parts[0], parts[1], g2, b2)
